# baseline (speedup 1.0000x reference)
import functools

import jax
import jax.numpy as jnp
from jax import lax
from jax.experimental import pallas as pl
from jax.experimental.pallas import tpu as pltpu
from jax.experimental.pallas import tpu_sc as plsc

B = 4096
V = 100000
D = 32
N_CAT_A = 26
N_CAT_B = 13

NC = 2
NS = 16
NW = NC * NS
BPW = B // NW

_mesh = plsc.VectorSubcoreMesh(core_axis_name="c", subcore_axis_name="s")


@functools.partial(
    pl.kernel,
    out_type=(
        jax.ShapeDtypeStruct((B, N_CAT_A, D), jnp.float32),
        jax.ShapeDtypeStruct((B, N_CAT_A, D), jnp.float32),
    ),
    mesh=_mesh,
    scratch_types=[
        pltpu.VMEM((BPW,), jnp.int32),
        pltpu.VMEM((BPW, D), jnp.float32),
        pltpu.VMEM((BPW,), jnp.float32),
        pltpu.VMEM((N_CAT_B, D), jnp.float32),
        pltpu.VMEM((N_CAT_B, D), jnp.float32),
        pltpu.SemaphoreType.DMA,
    ],
    compiler_params=pltpu.CompilerParams(use_tc_tiling_on_sc=False),
)
def _embed(embA_f, idxAT, embB_f, idxBT, numT, linW, linB, outA, outB,
           idxv, rowv, numv, wv, bv, sem):
    wid = lax.axis_index("s") * NC + lax.axis_index("c")
    b0 = wid * BPW

    def gather_col(c, emb_f, idxT, out):
        pltpu.sync_copy(idxT.at[c, pl.ds(b0, BPW)], idxv)
        off = c * V
        for j in range(BPW // 16):
            sl = pl.ds(j * 16, 16)
            idxv[sl] = idxv[sl] + off
        pltpu.async_copy(emb_f.at[idxv], rowv, sem).wait()
        pltpu.sync_copy(rowv, out.at[pl.ds(b0, BPW), c])

    def body_a(c, carry):
        gather_col(c, embA_f, idxAT, outA)
        return carry

    lax.fori_loop(0, N_CAT_A, body_a, 0)

    def body_b(c, carry):
        gather_col(c, embB_f, idxBT, outB)
        return carry

    lax.fori_loop(0, N_CAT_B, body_b, 0)

    pltpu.sync_copy(linW, wv)
    pltpu.sync_copy(linB, bv)

    def body_num(c, carry):
        pltpu.sync_copy(numT.at[c, pl.ds(b0, BPW)], numv)
        w0 = wv[c, pl.ds(0, 16)]
        w1 = wv[c, pl.ds(16, 16)]
        v0 = bv[c, pl.ds(0, 16)]
        v1 = bv[c, pl.ds(16, 16)]

        def body_row(bb, inner):
            nums = numv[pl.ds(bb * 16, 16)]
            for k in range(16):
                s = nums[k]
                rowv[bb * 16 + k, pl.ds(0, 16)] = s * w0 + v0
                rowv[bb * 16 + k, pl.ds(16, 16)] = s * w1 + v1
            return inner

        lax.fori_loop(0, BPW // 16, body_row, 0)
        pltpu.sync_copy(rowv, outB.at[pl.ds(b0, BPW), N_CAT_B + c])
        return carry

    lax.fori_loop(0, N_CAT_B, body_num, 0)


def kernel(tableA_cat, tableB_cat, tableB_num, embA, embB, linW, linB):
    embA_f = embA.reshape(N_CAT_A * V, D)
    embB_f = embB.reshape(N_CAT_B * V, D)
    idxAT = tableA_cat.astype(jnp.int32).T
    idxBT = tableB_cat.astype(jnp.int32).T
    numT = tableB_num.T
    outA, outB = _embed(embA_f, idxAT, embB_f, idxBT, numT, linW, linB)
    return (outA, outB)


# speedup vs baseline: 1.0109x; 1.0109x over previous
import functools

import jax
import jax.numpy as jnp
from jax import lax
from jax.experimental import pallas as pl
from jax.experimental.pallas import tpu as pltpu
from jax.experimental.pallas import tpu_sc as plsc

B = 4096
V = 100000
D = 32
N_CAT_A = 26
N_CAT_B = 13

NC = 2
NS = 16
NW = NC * NS
BPW = B // NW

_mesh = plsc.VectorSubcoreMesh(core_axis_name="c", subcore_axis_name="s")


@functools.partial(
    pl.kernel,
    out_type=(
        jax.ShapeDtypeStruct((B, N_CAT_A, D), jnp.float32),
        jax.ShapeDtypeStruct((B, N_CAT_A, D), jnp.float32),
    ),
    mesh=_mesh,
    scratch_types=[
        pltpu.VMEM((BPW, N_CAT_A), jnp.int32),
        pltpu.VMEM((BPW, N_CAT_B), jnp.int32),
        pltpu.VMEM((BPW, N_CAT_B), jnp.float32),
        pltpu.VMEM((N_CAT_B, D), jnp.float32),
        pltpu.VMEM((N_CAT_B, D), jnp.float32),
        pltpu.VMEM((BPW,), jnp.int32),
        pltpu.VMEM((BPW, D), jnp.float32),
        pltpu.SemaphoreType.DMA,
    ],
    compiler_params=pltpu.CompilerParams(
        use_tc_tiling_on_sc=False, needs_layout_passes=False),
)
def _embed(embA, tabA, embB, tabB, num, linW, linB, outA, outB,
           idxblkA, idxblkB, numblk, wv, bv, idxv, rowv, sem):
    wid = lax.axis_index("s") * NC + lax.axis_index("c")
    b0 = wid * BPW

    pltpu.sync_copy(tabA.at[pl.ds(b0, BPW), :], idxblkA)
    pltpu.sync_copy(tabB.at[pl.ds(b0, BPW), :], idxblkB)
    pltpu.sync_copy(num.at[pl.ds(b0, BPW), :], numblk)
    pltpu.sync_copy(linW, wv)
    pltpu.sync_copy(linB, bv)

    iota16 = lax.iota(jnp.int32, 16)

    def gather_col(c, emb, idxblk, out):
        c16 = jnp.full((16,), c, dtype=jnp.int32)
        for bb in range(BPW // 16):
            i16 = bb * 16 + iota16
            idxv[pl.ds(bb * 16, 16)] = plsc.load_gather(idxblk, [i16, c16])
        pltpu.async_copy(emb.at[c].at[idxv], rowv, sem).wait()
        pltpu.sync_copy(rowv, out.at[pl.ds(b0, BPW), c])

    def body_a(c, carry):
        gather_col(c, embA, idxblkA, outA)
        return carry

    lax.fori_loop(0, N_CAT_A, body_a, 0)

    def body_b(c, carry):
        gather_col(c, embB, idxblkB, outB)
        return carry

    lax.fori_loop(0, N_CAT_B, body_b, 0)

    def body_num(c, carry):
        c16 = jnp.full((16,), c, dtype=jnp.int32)
        w0 = wv[c, pl.ds(0, 16)]
        w1 = wv[c, pl.ds(16, 16)]
        v0 = bv[c, pl.ds(0, 16)]
        v1 = bv[c, pl.ds(16, 16)]

        def body_row(bb, inner):
            nums = plsc.load_gather(numblk, [bb * 16 + iota16, c16])
            for k in range(16):
                s = nums[k]
                rowv[bb * 16 + k, pl.ds(0, 16)] = s * w0 + v0
                rowv[bb * 16 + k, pl.ds(16, 16)] = s * w1 + v1
            return inner

        lax.fori_loop(0, BPW // 16, body_row, 0)
        pltpu.sync_copy(rowv, outB.at[pl.ds(b0, BPW), N_CAT_B + c])
        return carry

    lax.fori_loop(0, N_CAT_B, body_num, 0)


def kernel(tableA_cat, tableB_cat, tableB_num, embA, embB, linW, linB):
    outA, outB = _embed(embA, tableA_cat.astype(jnp.int32), embB,
                        tableB_cat.astype(jnp.int32), tableB_num, linW, linB)
    return (outA, outB)
